# baseline (speedup 1.0000x reference)
import functools

import jax
import jax.numpy as jnp
from jax import lax
from jax.experimental import pallas as pl
from jax.experimental.pallas import tpu as pltpu
from jax.experimental.pallas import tpu_sc as plsc

_NB = 16384
_NS = 50
_D = 32
_V = 1000000
_NW = 32
_BT = _NB // 128
_N_BLOCKS = _NS * _BT
_PER_W = _N_BLOCKS // _NW
_IDX_PER_W = _PER_W * 128


def _make_sc_gather():
  mesh = plsc.VectorSubcoreMesh(core_axis_name="c", subcore_axis_name="s")

  @functools.partial(
      pl.kernel,
      mesh=mesh,
      out_type=jax.ShapeDtypeStruct((_NS, _D // 8, _BT, 8, 128), jnp.float32),
      compiler_params=pltpu.CompilerParams(
          use_tc_tiling_on_sc=False, needs_layout_passes=False),
      scratch_types=[
          pltpu.VMEM((_IDX_PER_W,), jnp.int32),
          pltpu.VMEM((3, 256, _D), jnp.float32),
          pltpu.VMEM((2, 2, _D, 129), jnp.float32),
          pltpu.SemaphoreType.DMA,
          pltpu.SemaphoreType.DMA((2,)),
      ],
  )
  def gather_kernel(table_hbm, idx_hbm, out_hbm, idx_v, rows_v, tiles_v,
                    gsem, ssem):
    wid = lax.axis_index("s") * 2 + lax.axis_index("c")
    pltpu.sync_copy(idx_hbm.at[wid], idx_v)
    k_base = wid * _PER_W
    n_pairs = _PER_W // 2

    def gather_desc(g, buf):
      return pltpu.make_async_copy(
          table_hbm.at[idx_v.at[pl.ds(g * 256, 256)]],
          rows_v.at[buf],
          gsem,
      )

    def store_descs(g, buf):
      k = k_base + 2 * g
      s = k // _BT
      bt = k % _BT
      return [
          pltpu.make_async_copy(
              tiles_v.at[buf, pl.ds(0, 2), pl.ds(dt * 8, 8), pl.ds(0, 128)],
              out_hbm.at[s, dt, pl.ds(bt, 2)],
              ssem.at[buf],
          )
          for dt in range(_D // 8)
      ]

    for p in range(2):
      gather_desc(p, p % 3).start()

    def body(g, carry):
      rbuf = g % 3
      buf = g % 2
      gather_desc(g, rbuf).wait()
      @pl.when(g + 2 < n_pairs)
      def _():
        gather_desc(g + 2, (g + 2) % 3).start()
      @pl.when(g >= 2)
      def _():
        for c in store_descs(g - 2, buf):
          c.wait()

      lanes = lax.iota(jnp.int32, 16)
      for p in range(2):
        for j in range(128):
          col = jnp.full((16,), j, jnp.int32)
          for dd in range(2):
            vals = rows_v[rbuf, 128 * p + j, pl.ds(16 * dd, 16)]
            plsc.store_scatter(
                tiles_v.at[buf, p], [lanes + (16 * dd), col], vals)

      for c in store_descs(g, buf):
        c.start()
      return carry

    lax.fori_loop(0, n_pairs, body, 0)

    for g in (n_pairs - 2, n_pairs - 1):
      for c in store_descs(g, g % 2):
        c.wait()

  return gather_kernel


_sc_gather = _make_sc_gather()


def kernel(indices, word_vectors):
  idx_t = indices.astype(jnp.int32).T.reshape(_NW, _IDX_PER_W)
  y5 = _sc_gather(word_vectors, idx_t)
  out = jnp.transpose(y5, (2, 4, 0, 1, 3)).reshape(_NB, _NS, _D)
  return out


# speedup vs baseline: 1.0473x; 1.0473x over previous
import functools

import jax
import jax.numpy as jnp
from jax import lax
from jax.experimental import pallas as pl
from jax.experimental.pallas import tpu as pltpu
from jax.experimental.pallas import tpu_sc as plsc

_NB = 16384
_NS = 50
_D = 32
_V = 1000000
_NW = 32
_BT = _NB // 128
_N_BLOCKS = _NS * _BT
_PER_W = _N_BLOCKS // _NW
_IDX_PER_W = _PER_W * 128


def _make_sc_gather():
  mesh = plsc.VectorSubcoreMesh(core_axis_name="c", subcore_axis_name="s")

  @functools.partial(
      pl.kernel,
      mesh=mesh,
      out_type=jax.ShapeDtypeStruct((_NS, _D // 8, _BT, 8, 128), jnp.float32),
      compiler_params=pltpu.CompilerParams(
          use_tc_tiling_on_sc=False, needs_layout_passes=False),
      scratch_types=[
          pltpu.VMEM((_IDX_PER_W,), jnp.int32),
          pltpu.VMEM((4, 128, _D), jnp.float32),
          pltpu.VMEM((2, _D, 129), jnp.float32),
          pltpu.SemaphoreType.DMA,
          pltpu.SemaphoreType.DMA((2,)),
      ],
  )
  def gather_kernel(table_hbm, idx_hbm, out_hbm, idx_v, rows_v, tiles_v,
                    gsem, ssem):
    wid = lax.axis_index("s") * 2 + lax.axis_index("c")
    pltpu.sync_copy(idx_hbm.at[wid], idx_v)
    k_base = wid * _PER_W

    def gather_desc(g, buf):
      return pltpu.make_async_copy(
          table_hbm.at[idx_v.at[pl.ds(g * 128, 128)]],
          rows_v.at[buf],
          gsem,
      )

    def store_descs(g, buf):
      k = k_base + g
      s = k // _BT
      bt = k % _BT
      return [
          pltpu.make_async_copy(
              tiles_v.at[buf, pl.ds(dt * 8, 8), pl.ds(0, 128)],
              out_hbm.at[s, dt, bt],
              ssem.at[buf],
          )
          for dt in range(_D // 8)
      ]

    for p in range(3):
      gather_desc(p, p % 4).start()

    def body(g, carry):
      rbuf = g % 4
      buf = g % 2
      gather_desc(g, rbuf).wait()
      @pl.when(g + 3 < _PER_W)
      def _():
        gather_desc(g + 3, (g + 3) % 4).start()
      @pl.when(g >= 2)
      def _():
        for c in store_descs(g - 2, buf):
          c.wait()

      lanes = lax.iota(jnp.int32, 16)
      for j in range(128):
        col = jnp.full((16,), j, jnp.int32)
        for dd in range(2):
          vals = rows_v[rbuf, j, pl.ds(16 * dd, 16)]
          plsc.store_scatter(tiles_v.at[buf], [lanes + (16 * dd), col], vals)

      for c in store_descs(g, buf):
        c.start()
      return carry

    lax.fori_loop(0, _PER_W, body, 0)

    for g in (_PER_W - 2, _PER_W - 1):
      for c in store_descs(g, g % 2):
        c.wait()

  return gather_kernel


_sc_gather = _make_sc_gather()


def kernel(indices, word_vectors):
  idx_t = indices.astype(jnp.int32).T.reshape(_NW, _IDX_PER_W)
  y5 = _sc_gather(word_vectors, idx_t)
  out = jnp.transpose(y5, (2, 4, 0, 1, 3)).reshape(_NB, _NS, _D)
  return out
